# baseline (speedup 1.0000x reference)
import functools

import jax
import jax.numpy as jnp
from jax import lax
from jax.experimental import pallas as pl
from jax.experimental.pallas import tpu as pltpu
from jax.experimental.pallas import tpu_sc as plsc

_E = 16
_T = 16384
_D = 2048
_BT = 512
_NBUF = 4
_NSTEPS = _T // _BT

_NC = 2
_NS = 16
_NW = _NC * _NS
_TPW = _T // _NW


def _mm_body(x_hbm, wt_ref, b_ref, o_ref, buf, sems):
    i = pl.program_id(0)

    @pl.when(i == 0)
    def _prime():
        for j in range(_NBUF - 1):
            pltpu.make_async_copy(
                x_hbm.at[pl.ds(j * _BT, _BT), :], buf.at[j], sems.at[j]
            ).start()

    slot = lax.rem(i, _NBUF)
    pltpu.make_async_copy(
        x_hbm.at[pl.ds(i * _BT, _BT), :], buf.at[slot], sems.at[slot]
    ).wait()

    nxt = i + _NBUF - 1

    @pl.when(nxt < _NSTEPS)
    def _fetch():
        nslot = lax.rem(nxt, _NBUF)
        pltpu.make_async_copy(
            x_hbm.at[pl.ds(nxt * _BT, _BT), :], buf.at[nslot], sems.at[nslot]
        ).start()

    o_ref[...] = (
        jnp.dot(buf[slot], wt_ref[...], preferred_element_type=jnp.float32)
        + b_ref[...]
    )


def _logits_tc(x, wt, b2):
    return pl.pallas_call(
        _mm_body,
        grid=(_NSTEPS,),
        in_specs=[
            pl.BlockSpec(memory_space=pl.ANY),
            pl.BlockSpec((_D, _E), lambda i: (0, 0)),
            pl.BlockSpec((1, _E), lambda i: (0, 0)),
        ],
        out_specs=pl.BlockSpec((_BT, _E), lambda i: (i, 0)),
        out_shape=jax.ShapeDtypeStruct((_T, _E), jnp.float32),
        scratch_shapes=[
            pltpu.VMEM((_NBUF, _BT, _D), jnp.float32),
            pltpu.SemaphoreType.DMA((_NBUF,)),
        ],
    )(x, wt, b2)


def _sc_gate(logits):
    mesh = plsc.VectorSubcoreMesh(core_axis_name="c", subcore_axis_name="s")

    @functools.partial(
        pl.kernel,
        mesh=mesh,
        out_type=jax.ShapeDtypeStruct((_T, _E), jnp.float32),
        scratch_types=[
            pltpu.VMEM((_TPW, _E), jnp.float32),
            pltpu.VMEM((_TPW, _E), jnp.float32),
        ],
        compiler_params=pltpu.CompilerParams(needs_layout_passes=False),
    )
    def k(logits_hbm, out_hbm, lv, ov):
        wid = lax.axis_index("s") * _NC + lax.axis_index("c")
        base = wid * _TPW
        pltpu.sync_copy(logits_hbm.at[pl.ds(base, _TPW)], lv)
        iota = lax.iota(jnp.int32, 16)
        neginf = jnp.float32(-jnp.inf)

        @plsc.parallel_loop(0, _TPW, unroll=4)
        def body(i):
            v = lv[i]
            keys, _ = plsc.sort_key_val(v, v, descending=True)
            m1 = keys[0]
            m2 = keys[1]
            i1 = plsc.all_reduce_ffs(v == m1)
            eq1 = iota == i1
            i2 = plsc.all_reduce_ffs((v == m2) & (~eq1))
            keep = eq1 | (iota == i2)
            mx = jnp.maximum(m1, 0.0)
            masked = jnp.where(keep, v, 0.0)
            e = jnp.exp(masked - mx)
            z = jnp.where(iota == 0, m1, jnp.where(iota == 1, m2, 0.0))
            ez = jnp.exp(z - mx)
            s = ez[0] + ez[1] + 14.0 * ez[2]
            ov[i] = e / s

        pltpu.sync_copy(ov, out_hbm.at[pl.ds(base, _TPW)])

    return k(logits)


def kernel(x, W, b):
    wt = W.T
    b2 = b.reshape(1, _E)
    logits = _logits_tc(x, wt, b2)
    return _sc_gate(logits)


# speedup vs baseline: 1.0088x; 1.0088x over previous
import functools

import jax
import jax.numpy as jnp
from jax import lax
from jax.experimental import pallas as pl
from jax.experimental.pallas import tpu as pltpu
from jax.experimental.pallas import tpu_sc as plsc

_E = 16
_T = 16384
_D = 2048
_BT = 512
_NBUF = 4
_NSTEPS = _T // _BT

_NC = 2
_NS = 16
_NW = _NC * _NS
_TPW = _T // _NW


def _mm_body(x_hbm, wt_ref, b_ref, o_ref, buf, sems):
    i = pl.program_id(0)

    @pl.when(i == 0)
    def _prime():
        for j in range(_NBUF - 1):
            pltpu.make_async_copy(
                x_hbm.at[pl.ds(j * _BT, _BT), :], buf.at[j], sems.at[j]
            ).start()

    slot = lax.rem(i, _NBUF)
    pltpu.make_async_copy(
        x_hbm.at[pl.ds(i * _BT, _BT), :], buf.at[slot], sems.at[slot]
    ).wait()

    nxt = i + _NBUF - 1

    @pl.when(nxt < _NSTEPS)
    def _fetch():
        nslot = lax.rem(nxt, _NBUF)
        pltpu.make_async_copy(
            x_hbm.at[pl.ds(nxt * _BT, _BT), :], buf.at[nslot], sems.at[nslot]
        ).start()

    o_ref[...] = (
        jnp.dot(buf[slot], wt_ref[...], preferred_element_type=jnp.float32)
        + b_ref[...]
    )


def _logits_tc(x, wt, b2):
    return pl.pallas_call(
        _mm_body,
        grid=(_NSTEPS,),
        in_specs=[
            pl.BlockSpec(memory_space=pl.ANY),
            pl.BlockSpec((_D, _E), lambda i: (0, 0)),
            pl.BlockSpec((1, _E), lambda i: (0, 0)),
        ],
        out_specs=pl.BlockSpec((_BT, _E), lambda i: (i, 0)),
        out_shape=jax.ShapeDtypeStruct((_T, _E), jnp.float32),
        scratch_shapes=[
            pltpu.VMEM((_NBUF, _BT, _D), jnp.float32),
            pltpu.SemaphoreType.DMA((_NBUF,)),
        ],
    )(x, wt, b2)


def _sc_gate(logits):
    mesh = plsc.VectorSubcoreMesh(core_axis_name="c", subcore_axis_name="s")

    @functools.partial(
        pl.kernel,
        mesh=mesh,
        out_type=jax.ShapeDtypeStruct((_T, _E), jnp.float32),
        scratch_types=[
            pltpu.VMEM((_TPW, _E), jnp.float32),
            pltpu.VMEM((_TPW, _E), jnp.float32),
        ],
        compiler_params=pltpu.CompilerParams(needs_layout_passes=False),
    )
    def k(logits_hbm, out_hbm, lv, ov):
        wid = lax.axis_index("s") * _NC + lax.axis_index("c")
        base = wid * _TPW
        pltpu.sync_copy(logits_hbm.at[pl.ds(base, _TPW)], lv)
        iota = lax.iota(jnp.int32, 16)
        neginf = jnp.float32(-jnp.inf)

        @plsc.parallel_loop(0, _TPW, unroll=4)
        def body(i):
            v = lv[i]
            m1 = jnp.max(v)
            i1 = plsc.all_reduce_ffs(v == m1)
            v2 = jnp.where(iota == i1, neginf, v)
            m2 = jnp.max(v2)
            i2 = plsc.all_reduce_ffs(v2 == m2)
            keep = (iota == i1) | (iota == i2)
            masked = jnp.where(keep, v, 0.0)
            e = jnp.exp(masked - jnp.maximum(m1, 0.0))
            ov[i] = e / jnp.sum(e)

        pltpu.sync_copy(ov, out_hbm.at[pl.ds(base, _TPW)])

    return k(logits)


def kernel(x, W, b):
    wt = W.T
    b2 = b.reshape(1, _E)
    logits = _logits_tc(x, wt, b2)
    return _sc_gate(logits)
